# baseline (speedup 1.0000x reference)
import jax
import jax.numpy as jnp
from jax.experimental import pallas as pl
from jax.experimental.pallas import tpu as pltpu

D_MODEL_ = 2048
N_EXP_ = 64
BM_ = 1024


def _router_body(x_ref, w_ref, b_ref, o_ref):
    acc = jax.lax.dot_general(
        x_ref[0],
        w_ref[...],
        (((1,), (1,)), ((), ())),
        preferred_element_type=jnp.float32,
    )
    o_ref[0] = acc + b_ref[...].reshape(1, N_EXP_)


def kernel(x, W, b):
    bsz, seq, d = x.shape
    grid = (bsz, seq // BM_)
    out = pl.pallas_call(
        _router_body,
        grid=grid,
        in_specs=[
            pl.BlockSpec((1, BM_, d), lambda i, j: (i, j, 0)),
            pl.BlockSpec((N_EXP_, d), lambda i, j: (0, 0)),
            pl.BlockSpec((N_EXP_,), lambda i, j: (0,)),
        ],
        out_specs=pl.BlockSpec((1, BM_, N_EXP_), lambda i, j: (i, j, 0)),
        out_shape=jax.ShapeDtypeStruct((bsz, seq, N_EXP_), jnp.float32),
        compiler_params=pltpu.CompilerParams(
            dimension_semantics=("arbitrary", "arbitrary"),
        ),
    )(x, W, b)
    return out


# speedup vs baseline: 1.1745x; 1.1745x over previous
import jax
import jax.numpy as jnp
from jax.experimental import pallas as pl
from jax.experimental.pallas import tpu as pltpu

D_MODEL_ = 2048
N_EXP_ = 64
BM_ = 1024


def _router_body(x_ref, w_ref, b_ref, o_ref):
    acc = jax.lax.dot_general(
        w_ref[...],
        x_ref[0],
        (((1,), (1,)), ((), ())),
        preferred_element_type=jnp.float32,
    )
    o_ref[0] = acc + b_ref[...].reshape(N_EXP_, 1)


def kernel(x, W, b):
    bsz, seq, d = x.shape
    grid = (bsz, seq // BM_)
    out = pl.pallas_call(
        _router_body,
        grid=grid,
        in_specs=[
            pl.BlockSpec((1, BM_, d), lambda i, j: (i, j, 0)),
            pl.BlockSpec((N_EXP_, d), lambda i, j: (0, 0)),
            pl.BlockSpec((N_EXP_,), lambda i, j: (0,)),
        ],
        out_specs=pl.BlockSpec((1, N_EXP_, BM_), lambda i, j: (i, 0, j)),
        out_shape=jax.ShapeDtypeStruct((bsz, N_EXP_, seq), jnp.float32),
        compiler_params=pltpu.CompilerParams(
            dimension_semantics=("arbitrary", "arbitrary"),
        ),
    )(x, W, b)
    return jnp.swapaxes(out, 1, 2)
